# baseline (speedup 1.0000x reference)
import numpy as np
import jax
import jax.numpy as jnp
from jax.experimental import pallas as pl
from jax.experimental.pallas import tpu as pltpu

_EPS = 1e-08
_T = 100
_K = 20
_N = 64
_L = 2048


def _alpha_bar_table(num_steps=_T, s=0.01):
    t = np.arange(0, num_steps + 1, dtype=np.float32)
    f_t = np.cos(np.pi / 2 * (t / num_steps + s) / (1 + s)) ** 2
    ab = f_t / (f_t[0] + _EPS)
    return np.asarray(ab, dtype=np.float32)


_AB_TABLE = _alpha_bar_table()


def _threefry_bits(cnt):
    ks0 = jnp.uint32(0)
    ks1 = jnp.uint32(42)
    ks2 = jnp.uint32(np.uint32(0 ^ 42 ^ 0x1BD11BDA))

    def rol(x, r):
        return (x << jnp.uint32(r)) | (x >> jnp.uint32(32 - r))

    def rounds(x0, x1, rots):
        for r in rots:
            x0 = x0 + x1
            x1 = rol(x1, r)
            x1 = x0 ^ x1
        return x0, x1

    r0 = (13, 15, 26, 6)
    r1 = (17, 29, 16, 24)
    x0 = jnp.zeros_like(cnt)
    x1 = cnt + ks1
    x0, x1 = rounds(x0, x1, r0)
    x0 = x0 + ks1
    x1 = x1 + ks2 + jnp.uint32(1)
    x0, x1 = rounds(x0, x1, r1)
    x0 = x0 + ks2
    x1 = x1 + ks0 + jnp.uint32(2)
    x0, x1 = rounds(x0, x1, r0)
    x0 = x0 + ks0
    x1 = x1 + ks1 + jnp.uint32(3)
    x0, x1 = rounds(x0, x1, r1)
    x0 = x0 + ks1
    x1 = x1 + ks2 + jnp.uint32(4)
    x0, x1 = rounds(x0, x1, r0)
    x0 = x0 + ks2
    x1 = x1 + ks0 + jnp.uint32(5)
    return x0 ^ x1


def _row_body(c0_ref, x_ref, m_ref, t_ref, ab_ref, th_ref, xp_ref):
    i = pl.program_id(0)
    a = ab_ref[t_ref[i]]

    p = c0_ref[0]
    s = jnp.sum(p, axis=0, keepdims=True)
    c0 = (p + 1e-12) / (s + 1e-12)

    x = x_ref[0]
    ki = jax.lax.broadcasted_iota(jnp.int32, (_K, _L), 0)
    onehot = (ki == x).astype(jnp.float32)
    dot = jnp.sum(c0 * onehot, axis=0, keepdims=True)

    theta = ((1.0 - a) / _K) * c0 + (a * dot) * onehot
    m = m_ref[0] != 0
    theta = jnp.where(m, theta, onehot)
    z = jnp.sum(theta, axis=0, keepdims=True) + 1e-12
    thn = theta / z
    th_ref[0] = thn

    li = jax.lax.broadcasted_iota(jnp.uint32, (_K, _L), 1)
    kiu = jax.lax.broadcasted_iota(jnp.uint32, (_K, _L), 0)
    base = jnp.uint32(_L * _K) * jnp.asarray(i, jnp.uint32)
    cnt = base + li * jnp.uint32(_K) + kiu
    bits = _threefry_bits(cnt)
    fb = (bits >> jnp.uint32(9)) | jnp.uint32(0x3F800000)
    u = jax.lax.bitcast_convert_type(fb, jnp.float32) - jnp.float32(1.0)
    tiny = jnp.float32(np.finfo(np.float32).tiny)
    u = jnp.maximum(tiny, u + tiny)
    v = -jnp.log(u)

    score = (thn + 1e-12) / v
    mx = jnp.max(score, axis=0, keepdims=True)
    cand = jnp.where(score == mx, ki, jnp.int32(_K))
    xp_ref[0] = jnp.min(cand, axis=0, keepdims=True)


def kernel(x_t, c0_pred, mask_generate, t):
    c0t = jnp.transpose(c0_pred, (0, 2, 1))
    xr = x_t.astype(jnp.int32).reshape(_N, 1, _L)
    mr = mask_generate.astype(jnp.int32).reshape(_N, 1, _L)
    ab = jnp.asarray(_AB_TABLE)

    theta_t, xp = pl.pallas_call(
        _row_body,
        grid=(_N,),
        in_specs=[
            pl.BlockSpec((1, _K, _L), lambda i: (i, 0, 0)),
            pl.BlockSpec((1, 1, _L), lambda i: (i, 0, 0)),
            pl.BlockSpec((1, 1, _L), lambda i: (i, 0, 0)),
            pl.BlockSpec(memory_space=pltpu.SMEM),
            pl.BlockSpec(memory_space=pltpu.SMEM),
        ],
        out_specs=[
            pl.BlockSpec((1, _K, _L), lambda i: (i, 0, 0)),
            pl.BlockSpec((1, 1, _L), lambda i: (i, 0, 0)),
        ],
        out_shape=[
            jax.ShapeDtypeStruct((_N, _K, _L), jnp.float32),
            jax.ShapeDtypeStruct((_N, 1, _L), jnp.int32),
        ],
    )(c0t, xr, mr, t.astype(jnp.int32), ab)

    theta = jnp.transpose(theta_t, (0, 2, 1))
    x_prev = xp.reshape(_N, _L)
    return (theta, x_prev)
